# Initial kernel scaffold; baseline (speedup 1.0000x reference)
import jax
import jax.numpy as jnp
from jax.experimental import pallas as pl


def kernel(x_grid, y_grid, pos_x, pos_y, height, width, background):
    raise NotImplementedError("write your pallas kernel here")


# speedup vs baseline: 64.2697x; 64.2697x over previous
import functools

import jax
import jax.numpy as jnp
from jax import lax
from jax.experimental import pallas as pl
from jax.experimental.pallas import tpu as pltpu
from jax.experimental.pallas import tpu_sc as plsc

_ROWS = 32
_NW = 32
_PASSES = 2
_CHUNK = 2000
_L = 16


def _build_sc_call(H, W, N):
    n_chunks = N // _CHUNK
    stripe_words = _ROWS * W
    mesh = plsc.VectorSubcoreMesh(core_axis_name="c", subcore_axis_name="s")

    @functools.partial(
        pl.kernel,
        mesh=mesh,
        compiler_params=pltpu.CompilerParams(needs_layout_passes=False),
        out_type=jax.ShapeDtypeStruct((H * W,), jnp.float32),
        scratch_types=[
            pltpu.VMEM((_CHUNK + _L,), jnp.float32),
            pltpu.VMEM((_CHUNK + _L,), jnp.float32),
            pltpu.VMEM((_CHUNK + _L,), jnp.float32),
            pltpu.VMEM((_CHUNK + _L,), jnp.float32),
            pltpu.VMEM((_CHUNK + _L,), jnp.int32),
            pltpu.VMEM((stripe_words,), jnp.float32),
            pltpu.VMEM((_L,), jnp.float32),
        ],
    )
    def sc_image(px_h, py_h, h_h, w_h, bg_h, out_h,
                 px_v, py_v, h_v, w_v, lst_v, acc_v, bg_v):
        wid = lax.axis_index("s") * 2 + lax.axis_index("c")

        iota_i = lax.iota(jnp.int32, _L)
        iota_f = iota_i.astype(jnp.float32)

        def _ws_chunk(c, mx):
            pltpu.sync_copy(w_h.at[pl.ds(c * _CHUNK, _CHUNK)],
                            w_v.at[pl.ds(0, _CHUNK)])

            def _mx_body(i, m):
                return jnp.maximum(m, w_v[pl.ds(i * _L, _L)])

            return lax.fori_loop(0, _CHUNK // _L, _mx_body, mx)

        mx = lax.fori_loop(0, n_chunks, _ws_chunk, jnp.zeros((_L,), jnp.float32))
        wm = mx[0]
        for l in range(1, _L):
            wm = jnp.maximum(wm, mx[l])
        wmax4 = wm * 4.0
        wsi = wmax4.astype(jnp.int32)
        ws = wsi + (wmax4 > wsi.astype(jnp.float32)).astype(jnp.int32)
        ws_mask = (iota_i >= 8 - ws) & (iota_i <= 8 + ws)
        ws8 = ws >= 8

        pltpu.sync_copy(bg_h.at[pl.ds(0, _L)], bg_v)
        bg_vec = bg_v[pl.ds(0, _L)]

        def _pass_body(p, _):
            sid = wid + _NW * p
            r0 = sid * _ROWS

            def _init_body(i, _):
                for u in range(8):
                    acc_v[pl.ds((i * 8 + u) * _L, _L)] = bg_vec
                return 0

            lax.fori_loop(0, stripe_words // (_L * 8), _init_body, 0)

            def _chunk_body(c, _):
                pltpu.sync_copy(px_h.at[pl.ds(c * _CHUNK, _CHUNK)],
                                px_v.at[pl.ds(0, _CHUNK)])
                pltpu.sync_copy(py_h.at[pl.ds(c * _CHUNK, _CHUNK)],
                                py_v.at[pl.ds(0, _CHUNK)])
                pltpu.sync_copy(h_h.at[pl.ds(c * _CHUNK, _CHUNK)],
                                h_v.at[pl.ds(0, _CHUNK)])
                pltpu.sync_copy(w_h.at[pl.ds(c * _CHUNK, _CHUNK)],
                                w_v.at[pl.ds(0, _CHUNK)])

                def _filt_body(i, ptr):
                    py16 = py_v[pl.ds(i * _L, _L)]
                    yi16 = py16.astype(jnp.int32)
                    m = (yi16 >= r0 - 8) & (yi16 <= r0 + _ROWS - 1 + 8)
                    keys = jnp.where(m, i * _L + iota_i,
                                     jnp.int32(0x7FFFFFFF))
                    lst_v[pl.ds(ptr, _L)] = lax.sort(keys)
                    cnt = plsc.all_reduce_population_count(m)[0]
                    return ptr + cnt

                n_match = lax.fori_loop(0, _CHUNK // _L, _filt_body, 0)

                def _peak_body(k, _):
                    j = lst_v[pl.ds(k, _L)][0]
                    px = px_v[pl.ds(j, _L)][0]
                    py = py_v[pl.ds(j, _L)][0]
                    hh = h_v[pl.ds(j, _L)][0]
                    wwv = w_v[pl.ds(j, _L)]
                    xi = px.astype(jnp.int32)
                    yi = py.astype(jnp.int32)
                    fx = px - xi.astype(jnp.float32)
                    fy = py - yi.astype(jnp.float32)
                    x0 = xi - 8
                    y0 = yi - 8 - r0
                    inv = (-0.5 / (wwv * wwv + 1e-20))[0]

                    dx = iota_f - (8.0 + fx)
                    ex = jnp.exp(dx * dx * inv)
                    d16 = jnp.where(iota_i == 0, 8.0 - fx, 8.0 - fy)
                    e16 = jnp.exp(d16 * d16 * inv)
                    ex16 = e16[0]

                    colv = x0 + iota_i
                    okc = (colv >= 0) & (colv < W) & ws_mask
                    col_c = jnp.clip(colv, 0, W - 1)
                    c16 = x0 + 16
                    ok16c = (c16 < W) & ws8
                    c16c = jnp.minimum(c16, W - 1)

                    dy = iota_f - (8.0 + fy)
                    ey = hh * jnp.exp(dy * dy * inv)
                    ey16 = hh * e16[1]

                    rowv = y0 + iota_i
                    okr = (rowv >= 0) & (rowv < _ROWS) & ws_mask
                    ey = jnp.where(okr, ey, 0.0)
                    row_c = jnp.clip(rowv, 0, _ROWS - 1)
                    r16 = y0 + 16
                    ok16r = (r16 >= 0) & (r16 < _ROWS) & ws8
                    ey16 = jnp.where(ok16r, ey16, 0.0)
                    r16c = jnp.clip(r16, 0, _ROWS - 1)

                    for r in range(17):
                        s = ey[r] if r < 16 else ey16
                        rc = jnp.clip(y0 + r, 0, _ROWS - 1)
                        base = rc * W
                        plsc.addupdate_scatter(acc_v, [col_c + base],
                                               ex * s, mask=okc)
                    basev = row_c * W
                    plsc.addupdate_scatter(acc_v, [basev + c16c],
                                           ey * ex16, mask=okr & ok16c)
                    corner_idx = jnp.full((_L,), r16c * W + c16c, jnp.int32)
                    corner_val = jnp.full((_L,), ey16 * ex16, jnp.float32)
                    plsc.addupdate_scatter(acc_v, [corner_idx], corner_val,
                                           mask=(iota_i == 0) & ok16r & ok16c)
                    return 0

                lax.fori_loop(0, n_match, _peak_body, 0)
                return 0

            lax.fori_loop(0, n_chunks, _chunk_body, 0)
            pltpu.sync_copy(acc_v, out_h.at[pl.ds(r0 * W, stripe_words)])
            return 0

        lax.fori_loop(0, _PASSES, _pass_body, 0)

    return sc_image


def kernel(x_grid, y_grid, pos_x, pos_y, height, width, background):
    H, W = x_grid.shape
    N = pos_x.shape[0]
    bg16 = jnp.full((_L,), background, jnp.float32)
    sc_image = _build_sc_call(H, W, N)
    out = sc_image(pos_x, pos_y, height, width, bg16)
    return out.reshape(H, W)


# speedup vs baseline: 82.1182x; 1.2777x over previous
import functools

import jax
import jax.numpy as jnp
from jax import lax
from jax.experimental import pallas as pl
from jax.experimental.pallas import tpu as pltpu
from jax.experimental.pallas import tpu_sc as plsc

_ROWS = 32
_NW = 32
_PASSES = 2
_CHUNK = 10000
_L = 16


def _build_sc_call(H, W, N):
    n_chunks = N // _CHUNK
    stripe_words = _ROWS * W
    mesh = plsc.VectorSubcoreMesh(core_axis_name="c", subcore_axis_name="s")

    @functools.partial(
        pl.kernel,
        mesh=mesh,
        compiler_params=pltpu.CompilerParams(needs_layout_passes=False),
        out_type=jax.ShapeDtypeStruct((H * W,), jnp.float32),
        scratch_types=[
            pltpu.VMEM((_CHUNK + _L,), jnp.float32),
            pltpu.VMEM((_CHUNK + _L,), jnp.float32),
            pltpu.VMEM((_CHUNK + _L,), jnp.float32),
            pltpu.VMEM((_CHUNK + _L,), jnp.float32),
            pltpu.VMEM((_CHUNK + _L,), jnp.int32),
            pltpu.VMEM((stripe_words,), jnp.float32),
            pltpu.VMEM((_L,), jnp.float32),
        ],
    )
    def sc_image(px_h, py_h, h_h, w_h, bg_h, out_h,
                 px_v, py_v, h_v, w_v, lst_v, acc_v, bg_v):
        wid = lax.axis_index("s") * 2 + lax.axis_index("c")

        iota_i = lax.iota(jnp.int32, _L)
        iota_f = iota_i.astype(jnp.float32)

        def _ws_chunk(c, mx):
            pltpu.sync_copy(w_h.at[pl.ds(c * _CHUNK, _CHUNK)],
                            w_v.at[pl.ds(0, _CHUNK)])

            def _mx_body(i, m):
                return jnp.maximum(m, w_v[pl.ds(i * _L, _L)])

            return lax.fori_loop(0, _CHUNK // _L, _mx_body, mx)

        mx = lax.fori_loop(0, n_chunks, _ws_chunk, jnp.zeros((_L,), jnp.float32))
        wm = mx[0]
        for l in range(1, _L):
            wm = jnp.maximum(wm, mx[l])
        wmax4 = wm * 4.0
        wsi = wmax4.astype(jnp.int32)
        ws = wsi + (wmax4 > wsi.astype(jnp.float32)).astype(jnp.int32)
        ws_mask = (iota_i >= 8 - ws) & (iota_i <= 8 + ws)
        ws8 = ws >= 8

        pltpu.sync_copy(bg_h.at[pl.ds(0, _L)], bg_v)
        bg_vec = bg_v[pl.ds(0, _L)]

        def _pass_body(p, _):
            sid = wid + _NW * p
            r0 = sid * _ROWS

            def _init_body(i, _):
                for u in range(8):
                    acc_v[pl.ds((i * 8 + u) * _L, _L)] = bg_vec
                return 0

            lax.fori_loop(0, stripe_words // (_L * 8), _init_body, 0)

            def _chunk_body(c, _):
                pltpu.sync_copy(px_h.at[pl.ds(c * _CHUNK, _CHUNK)],
                                px_v.at[pl.ds(0, _CHUNK)])
                pltpu.sync_copy(py_h.at[pl.ds(c * _CHUNK, _CHUNK)],
                                py_v.at[pl.ds(0, _CHUNK)])
                pltpu.sync_copy(h_h.at[pl.ds(c * _CHUNK, _CHUNK)],
                                h_v.at[pl.ds(0, _CHUNK)])
                pltpu.sync_copy(w_h.at[pl.ds(c * _CHUNK, _CHUNK)],
                                w_v.at[pl.ds(0, _CHUNK)])

                def _filt_body(i, ptr):
                    py16 = py_v[pl.ds(i * _L, _L)]
                    yi16 = py16.astype(jnp.int32)
                    m = (yi16 >= r0 - 8) & (yi16 <= r0 + _ROWS - 1 + 8)
                    keys = jnp.where(m, i * _L + iota_i,
                                     jnp.int32(0x7FFFFFFF))
                    lst_v[pl.ds(ptr, _L)] = lax.sort(keys)
                    cnt = plsc.all_reduce_population_count(m)[0]
                    return ptr + cnt

                n_match = lax.fori_loop(0, _CHUNK // _L, _filt_body, 0)

                def _peak_body(k, _):
                    j = lst_v[pl.ds(k, _L)][0]
                    px = px_v[pl.ds(j, _L)][0]
                    py = py_v[pl.ds(j, _L)][0]
                    hh = h_v[pl.ds(j, _L)][0]
                    wwv = w_v[pl.ds(j, _L)]
                    xi = px.astype(jnp.int32)
                    yi = py.astype(jnp.int32)
                    fx = px - xi.astype(jnp.float32)
                    fy = py - yi.astype(jnp.float32)
                    x0 = xi - 8
                    y0 = yi - 8 - r0
                    inv = (-0.5 / (wwv * wwv + 1e-20))[0]

                    dx = iota_f - (8.0 + fx)
                    ex = jnp.exp(dx * dx * inv)
                    d16 = jnp.where(iota_i == 0, 8.0 - fx, 8.0 - fy)
                    e16 = jnp.exp(d16 * d16 * inv)
                    ex16 = e16[0]

                    colv = x0 + iota_i
                    okc = (colv >= 0) & (colv < W) & ws_mask
                    col_c = jnp.clip(colv, 0, W - 1)
                    c16 = x0 + 16
                    ok16c = (c16 < W) & ws8
                    c16c = jnp.minimum(c16, W - 1)

                    dy = iota_f - (8.0 + fy)
                    ey = hh * jnp.exp(dy * dy * inv)
                    ey16 = hh * e16[1]

                    rowv = y0 + iota_i
                    okr = (rowv >= 0) & (rowv < _ROWS) & ws_mask
                    ey = jnp.where(okr, ey, 0.0)
                    row_c = jnp.clip(rowv, 0, _ROWS - 1)
                    r16 = y0 + 16
                    ok16r = (r16 >= 0) & (r16 < _ROWS) & ws8
                    ey16 = jnp.where(ok16r, ey16, 0.0)
                    r16c = jnp.clip(r16, 0, _ROWS - 1)

                    for r in range(17):
                        s = ey[r] if r < 16 else ey16
                        rc = jnp.clip(y0 + r, 0, _ROWS - 1)
                        base = rc * W
                        plsc.addupdate_scatter(acc_v, [col_c + base],
                                               ex * s, mask=okc)
                    basev = row_c * W
                    plsc.addupdate_scatter(acc_v, [basev + c16c],
                                           ey * ex16, mask=okr & ok16c)
                    corner_idx = jnp.full((_L,), r16c * W + c16c, jnp.int32)
                    corner_val = jnp.full((_L,), ey16 * ex16, jnp.float32)
                    plsc.addupdate_scatter(acc_v, [corner_idx], corner_val,
                                           mask=(iota_i == 0) & ok16r & ok16c)
                    return 0

                lax.fori_loop(0, n_match, _peak_body, 0)
                return 0

            lax.fori_loop(0, n_chunks, _chunk_body, 0)
            pltpu.sync_copy(acc_v, out_h.at[pl.ds(r0 * W, stripe_words)])
            return 0

        lax.fori_loop(0, _PASSES, _pass_body, 0)

    return sc_image


def kernel(x_grid, y_grid, pos_x, pos_y, height, width, background):
    H, W = x_grid.shape
    N = pos_x.shape[0]
    bg16 = jnp.full((_L,), background, jnp.float32)
    sc_image = _build_sc_call(H, W, N)
    out = sc_image(pos_x, pos_y, height, width, bg16)
    return out.reshape(H, W)
